# baseline (speedup 1.0000x reference)
import functools

import jax
import jax.numpy as jnp
from jax.experimental import pallas as pl
from jax.experimental.pallas import tpu as pltpu


def _knn_body(q_ref, s_ref, out_ref, *, n_way, g, hw):
    rows = g * hw
    qb = q_ref[...].reshape(rows, q_ref.shape[-1])
    rq = 1.0 / (jnp.sqrt(jnp.sum(qb * qb, axis=1, keepdims=True)) + 1e-8)
    per_class = []
    for j in range(n_way):
        sj = s_ref[j]
        rs = 1.0 / (jnp.sqrt(jnp.sum(sj * sj, axis=0, keepdims=True)) + 1e-8)
        raw = jnp.dot(qb, sj, preferred_element_type=jnp.float32)
        inner = raw * rq * rs
        m1 = jnp.max(inner, axis=1, keepdims=True)
        eq1 = inner == m1
        n1 = jnp.sum(eq1.astype(jnp.float32), axis=1, keepdims=True)
        s2 = jnp.where(eq1, -3.0, inner)
        m2 = jnp.max(s2, axis=1, keepdims=True)
        eq2 = s2 == m2
        n2 = jnp.sum(eq2.astype(jnp.float32), axis=1, keepdims=True)
        s3 = jnp.where(eq2, -3.0, s2)
        m3 = jnp.max(s3, axis=1, keepdims=True)
        t1 = jnp.minimum(n1, 3.0)
        t2 = jnp.clip(3.0 - n1, 0.0, n2)
        t3 = jnp.maximum(3.0 - n1 - n2, 0.0)
        per_class.append(m1 * t1 + m2 * t2 + m3 * t3)
    cat = jnp.concatenate(per_class, axis=1)
    seg = jax.lax.broadcasted_iota(jnp.int32, (g, rows), 1) // hw
    qid = jax.lax.broadcasted_iota(jnp.int32, (g, rows), 0)
    ind = (seg == qid).astype(jnp.float32)
    out_ref[...] = jnp.dot(ind, cat, preferred_element_type=jnp.float32)[None]


def kernel(q, S, av_num):
    B, C, h, w = q.shape
    n_way, _, M = S.shape
    hw = h * w
    G = 5
    qf = jnp.transpose(q.reshape(B, C, hw), (0, 2, 1))

    out = pl.pallas_call(
        functools.partial(_knn_body, n_way=n_way, g=G, hw=hw),
        grid=(B // G,),
        in_specs=[
            pl.BlockSpec((G, hw, C), lambda b: (b, 0, 0)),
            pl.BlockSpec((n_way, C, M), lambda b: (0, 0, 0)),
        ],
        out_specs=pl.BlockSpec((1, G, n_way), lambda b: (b, 0, 0)),
        out_shape=jax.ShapeDtypeStruct((B // G, G, n_way), jnp.float32),
        compiler_params=pltpu.CompilerParams(
            dimension_semantics=("parallel",),
        ),
    )(qf, S)
    out = out.reshape(B, n_way)
    return (out, out)


# speedup vs baseline: 1.3221x; 1.3221x over previous
import functools

import jax
import jax.numpy as jnp
from jax.experimental import pallas as pl
from jax.experimental.pallas import tpu as pltpu


def _snorm_body(s_ref, out_ref):
    s = s_ref[...]
    rs = 1.0 / (jnp.sqrt(jnp.sum(s * s, axis=1, keepdims=True)) + 1e-8)
    out_ref[...] = s * rs


def _knn_body(q_ref, s_ref, out_ref, *, n_way):
    qb = q_ref[0]
    rq = 1.0 / (jnp.sqrt(jnp.sum(qb * qb, axis=1, keepdims=True)) + 1e-8)
    qn = qb * rq
    per_class = []
    for j in range(n_way):
        inner = jnp.dot(qn, s_ref[j], preferred_element_type=jnp.float32)
        m1 = jnp.max(inner, axis=1, keepdims=True)
        eq1 = inner == m1
        n1 = jnp.sum(eq1.astype(jnp.float32), axis=1, keepdims=True)
        s2 = jnp.where(eq1, -3.0, inner)
        m2 = jnp.max(s2, axis=1, keepdims=True)
        eq2 = s2 == m2
        n2 = jnp.sum(eq2.astype(jnp.float32), axis=1, keepdims=True)
        s3 = jnp.where(eq2, -3.0, s2)
        m3 = jnp.max(s3, axis=1, keepdims=True)
        t1 = jnp.minimum(n1, 3.0)
        t2 = jnp.clip(3.0 - n1, 0.0, n2)
        t3 = jnp.maximum(3.0 - n1 - n2, 0.0)
        per_class.append(m1 * t1 + m2 * t2 + m3 * t3)
    cat = jnp.concatenate(per_class, axis=1)
    out_ref[...] = jnp.sum(cat, axis=0, keepdims=True)[None]


def kernel(q, S, av_num):
    B, C, h, w = q.shape
    n_way, _, M = S.shape
    hw = h * w
    qf = jnp.transpose(q.reshape(B, C, hw), (0, 2, 1))

    Sn = pl.pallas_call(
        _snorm_body,
        out_shape=jax.ShapeDtypeStruct(S.shape, jnp.float32),
    )(S)

    out = pl.pallas_call(
        functools.partial(_knn_body, n_way=n_way),
        grid=(B,),
        in_specs=[
            pl.BlockSpec((1, hw, C), lambda b: (b, 0, 0)),
            pl.BlockSpec((n_way, C, M), lambda b: (0, 0, 0)),
        ],
        out_specs=pl.BlockSpec((1, 1, n_way), lambda b: (b, 0, 0)),
        out_shape=jax.ShapeDtypeStruct((B, 1, n_way), jnp.float32),
        compiler_params=pltpu.CompilerParams(
            dimension_semantics=("parallel",),
        ),
    )(qf, Sn)
    out = out.reshape(B, n_way)
    return (out, out)


# speedup vs baseline: 2.2053x; 1.6681x over previous
import functools

import jax
import jax.numpy as jnp
from jax.experimental import pallas as pl
from jax.experimental.pallas import tpu as pltpu

_LANES = 128
_NEG = -3.0


def _snorm_body(s_ref, out_ref):
    s = s_ref[...]
    rs = 1.0 / (jnp.sqrt(jnp.sum(s * s, axis=1, keepdims=True)) + 1e-8)
    out_ref[...] = s * rs


def _knn_body(q_ref, s_ref, out_ref, *, n_way, m_real):
    qb = q_ref[0]
    rq = 1.0 / (jnp.sqrt(jnp.sum(qb * qb, axis=1, keepdims=True)) + 1e-8)
    qn = qb * rq
    hw = qb.shape[0]
    m_pad = s_ref.shape[-1]
    n_chunks = m_pad // _LANES
    n_real_last = m_real - _LANES * (n_chunks - 1)
    lane = jax.lax.broadcasted_iota(jnp.int32, (hw, _LANES), 1)
    per_class = []
    for j in range(n_way):
        inner = jnp.dot(qn, s_ref[j], preferred_element_type=jnp.float32)
        def chunk(c):
            v = inner[:, c * _LANES:(c + 1) * _LANES]
            if c == n_chunks - 1 and n_real_last < _LANES:
                v = jnp.where(lane < n_real_last, v, _NEG)
            return v
        a = chunk(0)
        b = jnp.minimum(a, chunk(1))
        a = jnp.maximum(a, chunk(1))
        for c in range(2, n_chunks):
            v = chunk(c)
            a2 = jnp.maximum(a, v)
            t = jnp.minimum(a, v)
            b2 = jnp.maximum(b, t)
            u = jnp.minimum(b, t)
            if c == 2:
                cc = u
            else:
                cc = jnp.maximum(cc, u)
            a, b = a2, b2
        cand = jnp.concatenate([a, b, cc], axis=1)
        m1 = jnp.max(cand, axis=1, keepdims=True)
        eq1 = cand == m1
        n1 = jnp.sum(eq1.astype(jnp.float32), axis=1, keepdims=True)
        s2 = jnp.where(eq1, _NEG, cand)
        m2 = jnp.max(s2, axis=1, keepdims=True)
        eq2 = s2 == m2
        n2 = jnp.sum(eq2.astype(jnp.float32), axis=1, keepdims=True)
        s3 = jnp.where(eq2, _NEG, s2)
        m3 = jnp.max(s3, axis=1, keepdims=True)
        t1 = jnp.minimum(n1, 3.0)
        t2 = jnp.clip(3.0 - n1, 0.0, n2)
        t3 = jnp.maximum(3.0 - n1 - n2, 0.0)
        per_class.append(m1 * t1 + m2 * t2 + m3 * t3)
    cat = jnp.concatenate(per_class, axis=1)
    out_ref[...] = jnp.sum(cat, axis=0, keepdims=True)[None]


def kernel(q, S, av_num):
    B, C, h, w = q.shape
    n_way, _, M = S.shape
    hw = h * w
    m_pad = ((M + _LANES - 1) // _LANES) * _LANES
    qf = jnp.transpose(q.reshape(B, C, hw), (0, 2, 1))
    Sp = jnp.pad(S, ((0, 0), (0, 0), (0, m_pad - M)))

    Sn = pl.pallas_call(
        _snorm_body,
        out_shape=jax.ShapeDtypeStruct(Sp.shape, jnp.float32),
    )(Sp)

    out = pl.pallas_call(
        functools.partial(_knn_body, n_way=n_way, m_real=M),
        grid=(B,),
        in_specs=[
            pl.BlockSpec((1, hw, C), lambda b: (b, 0, 0)),
            pl.BlockSpec((n_way, C, m_pad), lambda b: (0, 0, 0)),
        ],
        out_specs=pl.BlockSpec((1, 1, n_way), lambda b: (b, 0, 0)),
        out_shape=jax.ShapeDtypeStruct((B, 1, n_way), jnp.float32),
        compiler_params=pltpu.CompilerParams(
            dimension_semantics=("parallel",),
        ),
    )(qf, Sn)
    out = out.reshape(B, n_way)
    return (out, out)


# speedup vs baseline: 2.2311x; 1.0117x over previous
import functools

import jax
import jax.numpy as jnp
from jax.experimental import pallas as pl
from jax.experimental.pallas import tpu as pltpu

_LANES = 128
_NEG = -3.0


def _snorm_body(s_ref, out_ref, *, m_real):
    c = pl.program_id(0)
    s = s_ref[...]
    col = c * _LANES + jax.lax.broadcasted_iota(jnp.int32, s.shape, 2)
    s = jnp.where(col < m_real, s, 0.0)
    rs = 1.0 / (jnp.sqrt(jnp.sum(s * s, axis=1, keepdims=True)) + 1e-8)
    out_ref[...] = s * rs


def _knn_body(q_ref, s_ref, out_ref, *, n_way, m_real):
    qb = q_ref[0]
    rq = 1.0 / (jnp.sqrt(jnp.sum(qb * qb, axis=0, keepdims=True)) + 1e-8)
    qn = qb * rq
    hw = qb.shape[1]
    m_pad = s_ref.shape[-1]
    n_chunks = m_pad // _LANES
    n_real_last = m_real - _LANES * (n_chunks - 1)
    lane = jax.lax.broadcasted_iota(jnp.int32, (hw, _LANES), 1)
    per_class = []
    for j in range(n_way):
        inner = jax.lax.dot_general(
            qn, s_ref[j],
            dimension_numbers=(((0,), (0,)), ((), ())),
            preferred_element_type=jnp.float32,
        )
        def chunk(c):
            v = inner[:, c * _LANES:(c + 1) * _LANES]
            if c == n_chunks - 1 and n_real_last < _LANES:
                v = jnp.where(lane < n_real_last, v, _NEG)
            return v
        a = chunk(0)
        b = jnp.minimum(a, chunk(1))
        a = jnp.maximum(a, chunk(1))
        for c in range(2, n_chunks):
            v = chunk(c)
            a2 = jnp.maximum(a, v)
            t = jnp.minimum(a, v)
            b2 = jnp.maximum(b, t)
            u = jnp.minimum(b, t)
            if c == 2:
                cc = u
            else:
                cc = jnp.maximum(cc, u)
            a, b = a2, b2
        cand = jnp.concatenate([a, b, cc], axis=1)
        m1 = jnp.max(cand, axis=1, keepdims=True)
        eq1 = cand == m1
        n1 = jnp.sum(eq1.astype(jnp.float32), axis=1, keepdims=True)
        s2 = jnp.where(eq1, _NEG, cand)
        m2 = jnp.max(s2, axis=1, keepdims=True)
        eq2 = s2 == m2
        n2 = jnp.sum(eq2.astype(jnp.float32), axis=1, keepdims=True)
        s3 = jnp.where(eq2, _NEG, s2)
        m3 = jnp.max(s3, axis=1, keepdims=True)
        t1 = jnp.minimum(n1, 3.0)
        t2 = jnp.clip(3.0 - n1, 0.0, n2)
        t3 = jnp.maximum(3.0 - n1 - n2, 0.0)
        per_class.append(m1 * t1 + m2 * t2 + m3 * t3)
    cat = jnp.concatenate(per_class, axis=1)
    out_ref[...] = jnp.sum(cat, axis=0, keepdims=True)[None]


def kernel(q, S, av_num):
    B, C, h, w = q.shape
    n_way, _, M = S.shape
    hw = h * w
    m_pad = ((M + _LANES - 1) // _LANES) * _LANES
    n_chunks = m_pad // _LANES
    qf = q.reshape(B, C, hw)

    Sn = pl.pallas_call(
        functools.partial(_snorm_body, m_real=M),
        grid=(n_chunks,),
        in_specs=[pl.BlockSpec((n_way, C, _LANES), lambda c: (0, 0, c))],
        out_specs=pl.BlockSpec((n_way, C, _LANES), lambda c: (0, 0, c)),
        out_shape=jax.ShapeDtypeStruct((n_way, C, m_pad), jnp.float32),
        compiler_params=pltpu.CompilerParams(
            dimension_semantics=("parallel",),
        ),
    )(S)

    out = pl.pallas_call(
        functools.partial(_knn_body, n_way=n_way, m_real=M),
        grid=(B,),
        in_specs=[
            pl.BlockSpec((1, C, hw), lambda b: (b, 0, 0)),
            pl.BlockSpec((n_way, C, m_pad), lambda b: (0, 0, 0)),
        ],
        out_specs=pl.BlockSpec((1, 1, n_way), lambda b: (b, 0, 0)),
        out_shape=jax.ShapeDtypeStruct((B, 1, n_way), jnp.float32),
        compiler_params=pltpu.CompilerParams(
            dimension_semantics=("parallel",),
        ),
    )(qf, Sn)
    out = out.reshape(B, n_way)
    return (out, out)
